# baseline (speedup 1.0000x reference)
import functools

import jax
import jax.numpy as jnp
from jax import lax
from jax.experimental import pallas as pl
from jax.experimental.pallas import tpu as pltpu
from jax.experimental.pallas import tpu_sc as plsc

N = 10000
D = 128
E = 320000

NC = 2
NS = 16
CH = 80
EPT = E // NS
NCHUNK = EPT // CH
ROWB = 640
ROWCH = 80

BK = 2000
GRID = N // BK

def _mesh():
  return plsc.VectorSubcoreMesh(
      core_axis_name="c", subcore_axis_name="s", num_cores=NC,
      num_subcores=NS)


def _sc_scratch():
  scr = [pltpu.VMEM_SHARED((N, D), jnp.float32)]
  scr += [pltpu.VMEM((CH, D), jnp.float32)] * 4
  scr += [pltpu.VMEM((CH,), jnp.int32)] * 8
  scr += [pltpu.VMEM((CH,), jnp.int32)] * 8
  scr += [pltpu.SemaphoreType.DMA] * 16
  return scr


def _do_rel(sid, zeros_h, y_src, dst_h, src_h, out_h, scr):
  acc = scr[0]
  R = scr[1:5]
  SB = scr[5:13]
  DB = scr[13:21]
  IS = scr[21:29]
  GS = scr[29:33]
  SS = scr[33:37]
  rowbase = sid * ROWB
  rowlast = N - (NS - 1) * ROWB

  @pl.when(sid < NS - 1)
  def _():
    pltpu.sync_copy(zeros_h, acc.at[pl.ds(rowbase, ROWB)])

  @pl.when(sid == NS - 1)
  def _():
    pltpu.sync_copy(zeros_h.at[pl.ds(0, rowlast)],
                    acc.at[pl.ds(rowbase, rowlast)])

  plsc.subcore_barrier()

  def idx_start(c, sl):
    pltpu.async_copy(src_h.at[sid, c], SB[sl], IS[sl])
    pltpu.async_copy(dst_h.at[sid, c], DB[sl], IS[sl])

  def idx_wait(c, sl):
    pltpu.make_async_copy(src_h.at[sid, c], SB[sl], IS[sl]).wait()
    pltpu.make_async_copy(dst_h.at[sid, c], DB[sl], IS[sl]).wait()

  H = CH // 2

  def gather_start(c, sl8, p):
    pltpu.async_copy(y_src.at[SB[sl8].at[pl.ds(0, H)]],
                     R[p].at[pl.ds(0, H)], GS[p])
    pltpu.async_copy(y_src.at[SB[sl8].at[pl.ds(H, H)]],
                     R[p].at[pl.ds(H, H)], GS[p])

  def gather_wait(sl8, p):
    pltpu.make_async_copy(y_src.at[SB[sl8].at[pl.ds(0, H)]],
                          R[p].at[pl.ds(0, H)], GS[p]).wait()
    pltpu.make_async_copy(y_src.at[SB[sl8].at[pl.ds(H, H)]],
                          R[p].at[pl.ds(H, H)], GS[p]).wait()

  def scatter_drain(p):
    pltpu.make_async_copy(R[p], acc.at[DB[p]], SS[p]).wait()

  def maybe_when(cond, fn):
    if isinstance(cond, bool):
      if cond:
        fn()
    else:
      pl.when(cond)(fn)

  def step(j, b, prep=True):
    p = b % 4
    if prep:
      q = (b + 3) % 4
      maybe_when(j >= 1, lambda: scatter_drain(q))
      idx_wait(j + 3, (b + 3) % 8)
      gather_start(j + 3, (b + 3) % 8, q)
    gather_wait(b, p)
    pltpu.async_copy(R[p], acc.at[DB[b]], SS[p], add=True)
    maybe_when(j + 5 < NCHUNK,
               lambda: idx_start(j + 5, (b + 5) % 8))

  for c in range(5):
    idx_start(c, c)
  for c in range(3):
    idx_wait(c, c)
    gather_start(c, c, c)

  def round_body(k, _):
    for b in range(8):
      step(8 * k + b, b)
    return 0

  NR = (NCHUNK - 3) // 8
  lax.fori_loop(0, NR, round_body, 0)
  for j in range(8 * NR, NCHUNK):
    step(j, j % 8, prep=(j + 3 < NCHUNK))
  for p in range(4):
    scatter_drain(p)
  plsc.subcore_barrier()

  @pl.when(sid < NS - 1)
  def _():
    pltpu.sync_copy(acc.at[pl.ds(rowbase, ROWB)],
                    out_h.at[pl.ds(rowbase, ROWB)])

  @pl.when(sid == NS - 1)
  def _():
    pltpu.sync_copy(acc.at[pl.ds(rowbase, rowlast)],
                    out_h.at[pl.ds(rowbase, rowlast)])

  plsc.subcore_barrier()


@functools.lru_cache(maxsize=None)
def _sc_spmm4():
  @functools.partial(
      pl.kernel,
      out_type=[jax.ShapeDtypeStruct((N, D), jnp.float32)] * 4,
      mesh=_mesh(),
      scratch_types=_sc_scratch())
  def spmm4(*refs):
    (ya, yb, zz,
     daa, saa, dab, sab_, dba, sba, dbb, sbb,
     oaa, oab, oba, obb) = refs[:15]
    scr = refs[15:]
    cid = lax.axis_index("c")
    sid = lax.axis_index("s")

    @pl.when(cid == 0)
    def _():
      _do_rel(sid, zz, ya, daa, saa, oaa, scr)
      _do_rel(sid, zz, yb, dab, sab_, oab, scr)

    @pl.when(cid == 1)
    def _():
      _do_rel(sid, zz, ya, dba, sba, oba, scr)
      _do_rel(sid, zz, yb, dbb, sbb, obb, scr)

  return spmm4


@functools.lru_cache(maxsize=None)
def _sc_spmm2():
  @functools.partial(
      pl.kernel,
      out_type=[jax.ShapeDtypeStruct((N, D), jnp.float32)] * 2,
      mesh=_mesh(),
      scratch_types=_sc_scratch())
  def spmm2(*refs):
    ya, yb, zz, daa, saa, dab, sab_, oaa, oab = refs[:9]
    scr = refs[9:]
    cid = lax.axis_index("c")
    sid = lax.axis_index("s")

    @pl.when(cid == 0)
    def _():
      _do_rel(sid, zz, ya, daa, saa, oaa, scr)

    @pl.when(cid == 1)
    def _():
      _do_rel(sid, zz, yb, dab, sab_, oab, scr)

  return spmm2


def _mm(x, w):
  return jnp.dot(x, w, preferred_element_type=jnp.float32)


def _fc1_body(xa, xb, W1a, b1a, W1b, b1b, Wf, bf, ya, yb):
  for x, W1, b1, y in ((xa, W1a, b1a, ya), (xb, W1b, b1b, yb)):
    h = jnp.maximum(_mm(x[...], W1[...]) + b1[...], 0.0)
    y[...] = _mm(h, Wf[...]) + bf[...]


def _fc1_call(xa, xb, W1a, b1a, W1b, b1b, Wf, bf):
  row = pl.BlockSpec((BK, D), lambda i: (i, 0))
  full = pl.BlockSpec((D, D), lambda i: (0, 0))
  bias = pl.BlockSpec((1, D), lambda i: (0, 0))
  return pl.pallas_call(
      _fc1_body,
      grid=(GRID,),
      in_specs=[row, row, full, bias, full, bias, full, bias],
      out_specs=[row, row],
      out_shape=[jax.ShapeDtypeStruct((N, D), jnp.float32)] * 2,
  )(xa, xb, W1a, b1a, W1b, b1b, Wf, bf)


def _tail_body(ngroups, bases, dout, *refs):
  ph = pl.program_id(0)
  i = pl.program_id(1)
  a2r = refs[0]
  Wn = refs[1]
  bn = refs[2]
  for g in range(ngroups):
    h0, h1, y, W, b, q = refs[3 + g * 6:3 + (g + 1) * 6]
    out = refs[3 + ngroups * 6 + g]
    ws = refs[3 + ngroups * 7 + g]
    a20 = a2r[0, bases[g]]
    a21 = a2r[0, bases[g] + 1]
    y_ = y[...]
    o0 = h0[...] + a20 * y_
    o1 = h1[...] + a21 * y_

    @pl.when((ph == 0) & (i == 0))
    def _():
      ws[...] = jnp.zeros_like(ws)

    @pl.when(ph == 0)
    def _():
      vals = []
      for o in (o0, o1):
        s = jnp.tanh(_mm(o, W[...]) + b[...])
        vals.append(jnp.sum(s * q[...]))
      r = lax.broadcasted_iota(jnp.int32, (8, 128), 0)
      c = lax.broadcasted_iota(jnp.int32, (8, 128), 1)
      upd = (jnp.where((r == 0) & (c == 0), vals[0], 0.0)
             + jnp.where((r == 0) & (c == 1), vals[1], 0.0))
      ws[...] += upd

    @pl.when(ph == 1)
    def _():
      w = ws[0:1, 0:2] / float(N)
      m = jnp.max(w)
      e = jnp.exp(w - m)
      beta = e / jnp.sum(e)
      comb = jnp.maximum(o0 * beta[0, 0] + o1 * beta[0, 1], 0.0)
      out[...] = _mm(comb, Wn[...]) + bn[...]


def _tail_call(groups, bases, a2row, Wn, bn):
  ng = len(groups)
  dout = Wn.shape[1]
  row = pl.BlockSpec((BK, D), lambda p, i: (i, 0))
  full = pl.BlockSpec((D, D), lambda p, i: (0, 0))
  bias = pl.BlockSpec((1, D), lambda p, i: (0, 0))
  wspec = pl.BlockSpec((D, dout), lambda p, i: (0, 0))
  bspec = pl.BlockSpec((1, dout), lambda p, i: (0, 0))
  orow = pl.BlockSpec((BK, dout), lambda p, i: (i, 0))
  in_specs = [bias, wspec, bspec] + [row, row, row, full, bias, bias] * ng
  args = [a2row, Wn, bn] + [a for grp in groups for a in grp]
  return pl.pallas_call(
      functools.partial(_tail_body, ng, tuple(bases), dout),
      grid=(2, GRID),
      in_specs=in_specs,
      out_specs=[orow] * ng,
      out_shape=[jax.ShapeDtypeStruct((N, dout), jnp.float32)] * ng,
      scratch_shapes=[pltpu.VMEM((8, 128), jnp.float32)] * ng,
  )(*args)


def kernel(x_a, x_b, edge_index_aa, edge_index_ab, edge_index_ba,
           edge_index_bb, A2_aa, A2_ab, A2_ba, A2_bb,
           W1_a, b1_a, W1_b, b1_b, Wf0, bf0, Wf1, bf1, W2, b2,
           saW0a, sab0a, saq0a, saW0b, sab0b, saq0b,
           saW1a, sab1a, saq1a, saW1b, sab1b, saq1b):
  f32 = jnp.float32
  r1 = lambda v: v.reshape(1, -1).astype(f32)

  e3 = lambda v: v.astype(jnp.int32).reshape(NS, NCHUNK, CH)
  daa, saa = e3(edge_index_aa[0]), e3(edge_index_aa[1])
  dab, sab_ = e3(edge_index_ab[0]), e3(edge_index_ab[1])
  dba, sba = e3(edge_index_ba[0]), e3(edge_index_ba[1])
  dbb, sbb = e3(edge_index_bb[0]), e3(edge_index_bb[1])
  zz = jnp.zeros((ROWB, D), f32)
  a2row = jnp.zeros((1, 128), f32)
  for col, v in enumerate((A2_aa, A2_ab, A2_ba, A2_bb)):
    a2row = a2row.at[0, col].set(v.reshape(())[()])

  y0a, y0b = _fc1_call(x_a, x_b, W1_a, r1(b1_a), W1_b, r1(b1_b),
                       Wf0, r1(bf0))
  haa, hab, hba, hbb = _sc_spmm4()(
      y0a, y0b, zz, daa, saa, dab, sab_, dba, sba, dbb, sbb)
  y1a, y1b = _tail_call(
      [(haa, hab, y0a, saW0a, r1(sab0a), r1(saq0a)),
       (hba, hbb, y0b, saW0b, r1(sab0b), r1(saq0b))],
      [0, 2], a2row, Wf1, r1(bf1))
  haa1, hab1 = _sc_spmm2()(y1a, y1b, zz, daa, saa, dab, sab_)
  (out,) = _tail_call(
      [(haa1, hab1, y1a, saW1a, r1(sab1a), r1(saq1a))],
      [0], a2row, W2, r1(b2))
  return out


# speedup vs baseline: 1.0057x; 1.0057x over previous
import functools

import jax
import jax.numpy as jnp
from jax import lax
from jax.experimental import pallas as pl
from jax.experimental.pallas import tpu as pltpu
from jax.experimental.pallas import tpu_sc as plsc

N = 10000
D = 128
E = 320000

NC = 2
NS = 16
CH = 80
EPT = E // NS
NCHUNK = EPT // CH
ROWB = 640
ROWCH = 80

BK = 5000
GRID = N // BK

def _mesh():
  return plsc.VectorSubcoreMesh(
      core_axis_name="c", subcore_axis_name="s", num_cores=NC,
      num_subcores=NS)


def _sc_scratch():
  scr = [pltpu.VMEM_SHARED((N, D), jnp.float32)]
  scr += [pltpu.VMEM((CH, D), jnp.float32)] * 4
  scr += [pltpu.VMEM((CH,), jnp.int32)] * 8
  scr += [pltpu.VMEM((CH,), jnp.int32)] * 8
  scr += [pltpu.SemaphoreType.DMA] * 16
  return scr


def _do_rel(sid, zeros_h, y_src, dst_h, src_h, out_h, scr):
  acc = scr[0]
  R = scr[1:5]
  SB = scr[5:13]
  DB = scr[13:21]
  IS = scr[21:29]
  GS = scr[29:33]
  SS = scr[33:37]
  rowbase = sid * ROWB
  rowlast = N - (NS - 1) * ROWB

  @pl.when(sid < NS - 1)
  def _():
    pltpu.sync_copy(zeros_h, acc.at[pl.ds(rowbase, ROWB)])

  @pl.when(sid == NS - 1)
  def _():
    pltpu.sync_copy(zeros_h.at[pl.ds(0, rowlast)],
                    acc.at[pl.ds(rowbase, rowlast)])

  plsc.subcore_barrier()

  def idx_start(c, sl):
    pltpu.async_copy(src_h.at[sid, c], SB[sl], IS[sl])
    pltpu.async_copy(dst_h.at[sid, c], DB[sl], IS[sl])

  def idx_wait(c, sl):
    pltpu.make_async_copy(src_h.at[sid, c], SB[sl], IS[sl]).wait()
    pltpu.make_async_copy(dst_h.at[sid, c], DB[sl], IS[sl]).wait()

  def gather_start(c, sl8, p):
    pltpu.async_copy(y_src.at[SB[sl8]], R[p], GS[p])

  def gather_wait(sl8, p):
    pltpu.make_async_copy(y_src.at[SB[sl8]], R[p], GS[p]).wait()

  def scatter_drain(p):
    pltpu.make_async_copy(R[p], acc.at[DB[p]], SS[p]).wait()

  def maybe_when(cond, fn):
    if isinstance(cond, bool):
      if cond:
        fn()
    else:
      pl.when(cond)(fn)

  def step(j, b, prep=True):
    p = b % 4
    if prep:
      q = (b + 3) % 4
      maybe_when(j >= 1, lambda: scatter_drain(q))
      idx_wait(j + 3, (b + 3) % 8)
      gather_start(j + 3, (b + 3) % 8, q)
    gather_wait(b, p)
    pltpu.async_copy(R[p], acc.at[DB[b]], SS[p], add=True)
    maybe_when(j + 5 < NCHUNK,
               lambda: idx_start(j + 5, (b + 5) % 8))

  for c in range(5):
    idx_start(c, c)
  for c in range(3):
    idx_wait(c, c)
    gather_start(c, c, c)

  def round_body(k, _):
    for b in range(8):
      step(8 * k + b, b)
    return 0

  NR = (NCHUNK - 3) // 8
  lax.fori_loop(0, NR, round_body, 0)
  for j in range(8 * NR, NCHUNK):
    step(j, j % 8, prep=(j + 3 < NCHUNK))
  for p in range(4):
    scatter_drain(p)
  plsc.subcore_barrier()

  @pl.when(sid < NS - 1)
  def _():
    pltpu.sync_copy(acc.at[pl.ds(rowbase, ROWB)],
                    out_h.at[pl.ds(rowbase, ROWB)])

  @pl.when(sid == NS - 1)
  def _():
    pltpu.sync_copy(acc.at[pl.ds(rowbase, rowlast)],
                    out_h.at[pl.ds(rowbase, rowlast)])

  plsc.subcore_barrier()


@functools.lru_cache(maxsize=None)
def _sc_spmm4():
  @functools.partial(
      pl.kernel,
      out_type=[jax.ShapeDtypeStruct((N, D), jnp.float32)] * 4,
      mesh=_mesh(),
      scratch_types=_sc_scratch())
  def spmm4(*refs):
    (ya, yb, zz,
     daa, saa, dab, sab_, dba, sba, dbb, sbb,
     oaa, oab, oba, obb) = refs[:15]
    scr = refs[15:]
    cid = lax.axis_index("c")
    sid = lax.axis_index("s")

    @pl.when(cid == 0)
    def _():
      _do_rel(sid, zz, ya, daa, saa, oaa, scr)
      _do_rel(sid, zz, yb, dab, sab_, oab, scr)

    @pl.when(cid == 1)
    def _():
      _do_rel(sid, zz, ya, dba, sba, oba, scr)
      _do_rel(sid, zz, yb, dbb, sbb, obb, scr)

  return spmm4


@functools.lru_cache(maxsize=None)
def _sc_spmm2():
  @functools.partial(
      pl.kernel,
      out_type=[jax.ShapeDtypeStruct((N, D), jnp.float32)] * 2,
      mesh=_mesh(),
      scratch_types=_sc_scratch())
  def spmm2(*refs):
    ya, yb, zz, daa, saa, dab, sab_, oaa, oab = refs[:9]
    scr = refs[9:]
    cid = lax.axis_index("c")
    sid = lax.axis_index("s")

    @pl.when(cid == 0)
    def _():
      _do_rel(sid, zz, ya, daa, saa, oaa, scr)

    @pl.when(cid == 1)
    def _():
      _do_rel(sid, zz, yb, dab, sab_, oab, scr)

  return spmm2


def _mm(x, w):
  return jnp.dot(x, w, preferred_element_type=jnp.float32)


def _fc1_body(xa, xb, W1a, b1a, W1b, b1b, Wf, bf, ya, yb):
  for x, W1, b1, y in ((xa, W1a, b1a, ya), (xb, W1b, b1b, yb)):
    h = jnp.maximum(_mm(x[...], W1[...]) + b1[...], 0.0)
    y[...] = _mm(h, Wf[...]) + bf[...]


def _fc1_call(xa, xb, W1a, b1a, W1b, b1b, Wf, bf):
  row = pl.BlockSpec((BK, D), lambda i: (i, 0))
  full = pl.BlockSpec((D, D), lambda i: (0, 0))
  bias = pl.BlockSpec((1, D), lambda i: (0, 0))
  return pl.pallas_call(
      _fc1_body,
      grid=(GRID,),
      in_specs=[row, row, full, bias, full, bias, full, bias],
      out_specs=[row, row],
      out_shape=[jax.ShapeDtypeStruct((N, D), jnp.float32)] * 2,
  )(xa, xb, W1a, b1a, W1b, b1b, Wf, bf)


def _tail_body(ngroups, bases, dout, *refs):
  ph = pl.program_id(0)
  i = pl.program_id(1)
  a2r = refs[0]
  Wn = refs[1]
  bn = refs[2]
  for g in range(ngroups):
    h0, h1, y, W, b, q = refs[3 + g * 6:3 + (g + 1) * 6]
    out = refs[3 + ngroups * 6 + g]
    ws = refs[3 + ngroups * 7 + g]
    a20 = a2r[0, bases[g]]
    a21 = a2r[0, bases[g] + 1]
    y_ = y[...]
    o0 = h0[...] + a20 * y_
    o1 = h1[...] + a21 * y_

    @pl.when((ph == 0) & (i == 0))
    def _():
      ws[...] = jnp.zeros_like(ws)

    @pl.when(ph == 0)
    def _():
      vals = []
      for o in (o0, o1):
        s = jnp.tanh(_mm(o, W[...]) + b[...])
        vals.append(jnp.sum(s * q[...]))
      r = lax.broadcasted_iota(jnp.int32, (8, 128), 0)
      c = lax.broadcasted_iota(jnp.int32, (8, 128), 1)
      upd = (jnp.where((r == 0) & (c == 0), vals[0], 0.0)
             + jnp.where((r == 0) & (c == 1), vals[1], 0.0))
      ws[...] += upd

    @pl.when(ph == 1)
    def _():
      w = ws[0:1, 0:2] / float(N)
      m = jnp.max(w)
      e = jnp.exp(w - m)
      beta = e / jnp.sum(e)
      comb = jnp.maximum(o0 * beta[0, 0] + o1 * beta[0, 1], 0.0)
      out[...] = _mm(comb, Wn[...]) + bn[...]


def _tail_call(groups, bases, a2row, Wn, bn):
  ng = len(groups)
  dout = Wn.shape[1]
  row = pl.BlockSpec((BK, D), lambda p, i: (i, 0))
  full = pl.BlockSpec((D, D), lambda p, i: (0, 0))
  bias = pl.BlockSpec((1, D), lambda p, i: (0, 0))
  wspec = pl.BlockSpec((D, dout), lambda p, i: (0, 0))
  bspec = pl.BlockSpec((1, dout), lambda p, i: (0, 0))
  orow = pl.BlockSpec((BK, dout), lambda p, i: (i, 0))
  in_specs = [bias, wspec, bspec] + [row, row, row, full, bias, bias] * ng
  args = [a2row, Wn, bn] + [a for grp in groups for a in grp]
  return pl.pallas_call(
      functools.partial(_tail_body, ng, tuple(bases), dout),
      grid=(2, GRID),
      in_specs=in_specs,
      out_specs=[orow] * ng,
      out_shape=[jax.ShapeDtypeStruct((N, dout), jnp.float32)] * ng,
      scratch_shapes=[pltpu.VMEM((8, 128), jnp.float32)] * ng,
  )(*args)


def kernel(x_a, x_b, edge_index_aa, edge_index_ab, edge_index_ba,
           edge_index_bb, A2_aa, A2_ab, A2_ba, A2_bb,
           W1_a, b1_a, W1_b, b1_b, Wf0, bf0, Wf1, bf1, W2, b2,
           saW0a, sab0a, saq0a, saW0b, sab0b, saq0b,
           saW1a, sab1a, saq1a, saW1b, sab1b, saq1b):
  f32 = jnp.float32
  r1 = lambda v: v.reshape(1, -1).astype(f32)

  e3 = lambda v: v.astype(jnp.int32).reshape(NS, NCHUNK, CH)
  daa, saa = e3(edge_index_aa[0]), e3(edge_index_aa[1])
  dab, sab_ = e3(edge_index_ab[0]), e3(edge_index_ab[1])
  dba, sba = e3(edge_index_ba[0]), e3(edge_index_ba[1])
  dbb, sbb = e3(edge_index_bb[0]), e3(edge_index_bb[1])
  zz = jnp.zeros((ROWB, D), f32)
  a2row = jnp.zeros((1, 128), f32)
  for col, v in enumerate((A2_aa, A2_ab, A2_ba, A2_bb)):
    a2row = a2row.at[0, col].set(v.reshape(())[()])

  y0a, y0b = _fc1_call(x_a, x_b, W1_a, r1(b1_a), W1_b, r1(b1_b),
                       Wf0, r1(bf0))
  haa, hab, hba, hbb = _sc_spmm4()(
      y0a, y0b, zz, daa, saa, dab, sab_, dba, sba, dbb, sbb)
  y1a, y1b = _tail_call(
      [(haa, hab, y0a, saW0a, r1(sab0a), r1(saq0a)),
       (hba, hbb, y0b, saW0b, r1(sab0b), r1(saq0b))],
      [0, 2], a2row, Wf1, r1(bf1))
  haa1, hab1 = _sc_spmm2()(y1a, y1b, zz, daa, saa, dab, sab_)
  (out,) = _tail_call(
      [(haa1, hab1, y1a, saW1a, r1(sab1a), r1(saq1a))],
      [0], a2row, W2, r1(b2))
  return out
